# baseline (speedup 1.0000x reference)
import functools

import jax
import jax.numpy as jnp
from jax import lax
from jax.experimental import pallas as pl
from jax.experimental.pallas import tpu as pltpu
from jax.experimental.pallas import tpu_sc as plsc

B, S = 16384, 50
V, D = 1_000_000, 32

NC, NS = 2, 16
NW = NC * NS
ROWS_PER_W = B // NW
CB = 32
NCHUNKS = ROWS_PER_W // CB
BT = 128
NBT = B // BT
DBLK = D // 8
NST = 10


def _make_sc_lookup():
  mesh = plsc.VectorSubcoreMesh(core_axis_name="c", subcore_axis_name="s")

  @functools.partial(
      pl.kernel,
      mesh=mesh,
      compiler_params=pltpu.CompilerParams(
          use_tc_tiling_on_sc=False, needs_layout_passes=False),
      out_type=jax.ShapeDtypeStruct((S, DBLK, NBT, 8, BT), jnp.float32),
      scratch_types=[
          pltpu.VMEM((CB, S), jnp.int32),
          pltpu.VMEM((CB, S), jnp.int32),
          pltpu.VMEM((CB, S, D), jnp.float32),
          pltpu.VMEM((CB, S, D), jnp.float32),
          *[pltpu.VMEM((DBLK, 1, 8, CB), jnp.float32)
            for _ in range(NST)],
          pltpu.SemaphoreType.DMA,
          pltpu.SemaphoreType.DMA,
          pltpu.SemaphoreType.DMA,
          pltpu.SemaphoreType.DMA,
          *[pltpu.SemaphoreType.DMA for _ in range(NST)],
      ],
  )
  def lookup(idx_hbm, table_hbm, out_hbm, idx0, idx1, rows0, rows1,
             *rest):
    stage = rest[:NST]
    isem = rest[NST:NST + 2]
    gsem = rest[NST + 2:NST + 4]
    ssem = rest[NST + 4:]
    wid = lax.axis_index("s") * NC + lax.axis_index("c")
    b_base = wid * ROWS_PER_W
    idx_v = (idx0, idx1)
    rows_v = (rows0, rows1)

    lane = lax.iota(jnp.int32, 16)
    zero16 = jnp.zeros((16,), jnp.int32)
    dblk_c = [lane // 8, lane // 8 + 2]
    drow_c = lax.rem(lane, 8)

    def idx_copy(c, s):
      return pltpu.make_async_copy(
          idx_hbm.at[pl.ds(b_base + c * CB, CB)], idx_v[s], isem[s])

    def fire_gathers(s):
      def go(j, carry):
        pltpu.make_async_copy(
            table_hbm.at[idx_v[s].at[j]], rows_v[s].at[j], gsem[s]).start()
        return carry
      lax.fori_loop(0, CB, go, 0)

    def drain_gathers(s):
      def dr(j, carry):
        pltpu.make_async_copy(
            table_hbm.at[idx_v[s].at[j]], rows_v[s].at[j], gsem[s]).wait()
        return carry
      lax.fori_loop(0, CB, dr, 0)

    def stage_copies(c, s_, q):
      btile = (b_base + c * CB) // BT
      bcol0 = (b_base + c * CB) % BT
      return [
          pltpu.make_async_copy(
              stage[q],
              out_hbm.at[s_, pl.ds(0, DBLK), pl.ds(btile, 1), pl.ds(0, 8),
                         pl.ds(bcol0, CB)],
              ssem[q])
      ]

    def transpose_chunk(c, s):
      def per_group(p, carry):
        for q in range(NST):
          s_ = NST * p + q
          @pl.when(s_ >= NST)
          def _():
            for cp in stage_copies(c, s_ - NST, q):
              cp.wait()
          for bb in range(CB):
            bb_c = zero16 + bb
            for h in range(2):
              vals = rows_v[s][bb, s_, pl.ds(16 * h, 16)]
              plsc.store_scatter(
                  stage[q], [dblk_c[h], zero16, drow_c, bb_c], vals)
          for cp in stage_copies(c, s_, q):
            cp.start()
        return carry
      lax.fori_loop(0, S // NST, per_group, 0)
      for q in range(NST):
        for cp in stage_copies(c, S - NST + q, q):
          cp.wait()

    idx_copy(0, 0).start()
    idx_copy(1, 1).start()
    idx_copy(0, 0).wait()
    fire_gathers(0)

    def pair_body(i, carry):
      for s in range(2):
        c = 2 * i + s
        @pl.when(c + 1 < NCHUNKS)
        def _():
          idx_copy(c + 1, 1 - s).wait()
          fire_gathers(1 - s)
        drain_gathers(s)
        @pl.when(c + 2 < NCHUNKS)
        def _():
          idx_copy(c + 2, s).start()
        transpose_chunk(c, s)
      return carry

    lax.fori_loop(0, NCHUNKS // 2, pair_body, 0)

  return lookup


_sc_lookup = _make_sc_lookup()


@jax.jit
def kernel(token_ids, weight):
  out5 = _sc_lookup(token_ids.astype(jnp.int32), weight)
  return out5.transpose(2, 4, 0, 1, 3).reshape(B, S, D)


# speedup vs baseline: 1.0090x; 1.0090x over previous
import functools

import jax
import jax.numpy as jnp
from jax import lax
from jax.experimental import pallas as pl
from jax.experimental.pallas import tpu as pltpu
from jax.experimental.pallas import tpu_sc as plsc

B, S = 16384, 50
V, D = 1_000_000, 32

NC, NS = 2, 16
NW = NC * NS
ROWS_PER_W = B // NW
CB = 32
NCHUNKS = ROWS_PER_W // CB
BT = 128
NBT = B // BT
DBLK = D // 8


def _make_sc_lookup():
  mesh = plsc.VectorSubcoreMesh(core_axis_name="c", subcore_axis_name="s")

  @functools.partial(
      pl.kernel,
      mesh=mesh,
      compiler_params=pltpu.CompilerParams(
          use_tc_tiling_on_sc=False, needs_layout_passes=False),
      out_type=jax.ShapeDtypeStruct((S, DBLK, NBT, 8, BT), jnp.float32),
      scratch_types=[
          pltpu.VMEM((CB, S), jnp.int32),
          pltpu.VMEM((CB, S), jnp.int32),
          pltpu.VMEM((CB, S, D), jnp.float32),
          pltpu.VMEM((CB, S, D), jnp.float32),
          pltpu.VMEM((DBLK, 1, 8, CB), jnp.float32),
          pltpu.VMEM((DBLK, 1, 8, CB), jnp.float32),
          pltpu.SemaphoreType.DMA,
          pltpu.SemaphoreType.DMA,
          pltpu.SemaphoreType.DMA,
          pltpu.SemaphoreType.DMA,
          pltpu.SemaphoreType.DMA,
          pltpu.SemaphoreType.DMA,
      ],
  )
  def lookup(idx_hbm, table_hbm, out_hbm, idx0, idx1, rows0, rows1,
             st0, st1, isem0, isem1, gsem0, gsem1, ssem0, ssem1):
    wid = lax.axis_index("s") * NC + lax.axis_index("c")
    b_base = wid * ROWS_PER_W
    idx_v = (idx0, idx1)
    rows_v = (rows0, rows1)
    stage = (st0, st1)
    isem = (isem0, isem1)
    gsem = (gsem0, gsem1)
    ssem = (ssem0, ssem1)

    lane = lax.iota(jnp.int32, 16)
    zero16 = jnp.zeros((16,), jnp.int32)
    dblk_c = [lane // 8, lane // 8 + 2]
    drow_c = lax.rem(lane, 8)

    def idx_copy(c, s):
      return pltpu.make_async_copy(
          idx_hbm.at[pl.ds(b_base + c * CB, CB)], idx_v[s], isem[s])

    def fire_gathers(s):
      def go(j, carry):
        pltpu.make_async_copy(
            table_hbm.at[idx_v[s].at[j]], rows_v[s].at[j], gsem[s]).start()
        return carry
      lax.fori_loop(0, CB, go, 0)

    def drain_gathers(s):
      def dr(j, carry):
        pltpu.make_async_copy(
            table_hbm.at[idx_v[s].at[j]], rows_v[s].at[j], gsem[s]).wait()
        return carry
      lax.fori_loop(0, CB, dr, 0)

    def stage_copies(c, s_, q):
      btile = (b_base + c * CB) // BT
      bcol0 = (b_base + c * CB) % BT
      return [
          pltpu.make_async_copy(
              stage[q],
              out_hbm.at[s_, pl.ds(0, DBLK), pl.ds(btile, 1), pl.ds(0, 8),
                         pl.ds(bcol0, CB)],
              ssem[q])
      ]

    def transpose_chunk(c, s):
      def per_pair(p, carry):
        for q in range(2):
          s_ = 2 * p + q
          @pl.when(s_ >= 2)
          def _():
            for cp in stage_copies(c, s_ - 2, q):
              cp.wait()
          for bb in range(CB):
            bb_c = zero16 + bb
            for h in range(2):
              vals = rows_v[s][bb, s_, pl.ds(16 * h, 16)]
              plsc.store_scatter(
                  stage[q], [dblk_c[h], zero16, drow_c, bb_c], vals)
          for cp in stage_copies(c, s_, q):
            cp.start()
        return carry
      lax.fori_loop(0, S // 2, per_pair, 0)
      for cp in stage_copies(c, S - 2, 0):
        cp.wait()
      for cp in stage_copies(c, S - 1, 1):
        cp.wait()

    idx_copy(0, 0).start()
    idx_copy(1, 1).start()
    idx_copy(0, 0).wait()
    fire_gathers(0)

    def pair_body(i, carry):
      for s in range(2):
        c = 2 * i + s
        @pl.when(c + 1 < NCHUNKS)
        def _():
          idx_copy(c + 1, 1 - s).wait()
          fire_gathers(1 - s)
        drain_gathers(s)
        @pl.when(c + 2 < NCHUNKS)
        def _():
          idx_copy(c + 2, s).start()
        transpose_chunk(c, s)
      return carry

    lax.fori_loop(0, NCHUNKS // 2, pair_body, 0)

  return lookup


_sc_lookup = _make_sc_lookup()


@jax.jit
def kernel(token_ids, weight):
  out5 = _sc_lookup(token_ids.astype(jnp.int32), weight)
  return out5.transpose(2, 4, 0, 1, 3).reshape(B, S, D)
